# Initial kernel scaffold; baseline (speedup 1.0000x reference)
import jax
import jax.numpy as jnp
from jax.experimental import pallas as pl


def kernel(x, Wr, br, Ww, bw, Wt, bt, Wp, bp, Wo, bo):
    raise NotImplementedError("write your pallas kernel here")


# speedup vs baseline: 15.4899x; 15.4899x over previous
import functools

import jax
import jax.numpy as jnp
from jax.experimental import pallas as pl
from jax.experimental.pallas import tpu as pltpu

_T = 512


def _fold_body(wp_ref, wob_ref, bp_ref, bo_ref, wq_ref, bq_ref):
    wob = wob_ref[...]
    wq_ref[...] = jnp.dot(wp_ref[...], wob, preferred_element_type=jnp.float32)
    bq_ref[...] = (
        jnp.dot(bp_ref[...], wob, preferred_element_type=jnp.float32) + bo_ref[...]
    )


def _main_body(x_ref, wc_ref, bc_ref, woa_ref, wq_ref, bq_ref, o_ref, m_ref):
    M = wq_ref.shape[0]
    T = x_ref.shape[1]

    @pl.when(pl.program_id(1) == 0)
    def _():
        m_ref[...] = jnp.zeros_like(m_ref)

    xb = x_ref[0]
    g = jnp.dot(xb, wc_ref[...], preferred_element_type=jnp.float32) + bc_ref[...]
    rw = jax.nn.sigmoid(g[:, :M])
    ww = jax.nn.sigmoid(g[:, M:2 * M])
    nm = g[:, 2 * M:]

    a = 1.0 - ww
    b = ww * nm

    A, Bv = a, b
    k = 1
    while k < T:
        pad1 = jnp.ones((k, M), jnp.float32)
        pad0 = jnp.zeros((k, M), jnp.float32)
        A_sh = jnp.concatenate([pad1, A[: T - k]], axis=0)
        B_sh = jnp.concatenate([pad0, Bv[: T - k]], axis=0)
        Bv = A * B_sh + Bv
        A = A * A_sh
        k *= 2

    m_in = m_ref[...]
    A_prev = jnp.concatenate([jnp.ones((1, M), jnp.float32), A[: T - 1]], axis=0)
    B_prev = jnp.concatenate([jnp.zeros((1, M), jnp.float32), Bv[: T - 1]], axis=0)
    rm = rw * (A_prev * m_in + B_prev)
    m_ref[...] = A[T - 1:] * m_in + Bv[T - 1:]

    h = jnp.dot(xb, woa_ref[...], preferred_element_type=jnp.float32)
    h = h + jnp.dot(rm, wq_ref[...], preferred_element_type=jnp.float32)
    o_ref[0] = jnp.tanh(h + bq_ref[...])


@functools.partial(jax.jit, static_argnames=("interpret",))
def _run(x, Wr, br, Ww, bw, Wt, bt, Wp, bp, Wo, bo, interpret=False):
    B, S, D = x.shape
    M = Wr.shape[1]
    T = _T

    Wc = jnp.concatenate([Wr, Ww, Wt], axis=1)
    bc = jnp.concatenate([br, bw, bt]).reshape(1, 3 * M)
    WoA = Wo[:D]
    WoB = Wo[D:]

    Wq, bq = pl.pallas_call(
        _fold_body,
        out_shape=(
            jax.ShapeDtypeStruct((M, D), jnp.float32),
            jax.ShapeDtypeStruct((1, D), jnp.float32),
        ),
        interpret=interpret,
    )(Wp, WoB, bp.reshape(1, D), bo.reshape(1, D))

    out = pl.pallas_call(
        _main_body,
        out_shape=jax.ShapeDtypeStruct((B, S, D), jnp.float32),
        grid=(B, S // T),
        in_specs=[
            pl.BlockSpec((1, T, D), lambda bi, ti: (bi, ti, 0)),
            pl.BlockSpec((D, 3 * M), lambda bi, ti: (0, 0)),
            pl.BlockSpec((1, 3 * M), lambda bi, ti: (0, 0)),
            pl.BlockSpec((D, D), lambda bi, ti: (0, 0)),
            pl.BlockSpec((M, D), lambda bi, ti: (0, 0)),
            pl.BlockSpec((1, D), lambda bi, ti: (0, 0)),
        ],
        out_specs=pl.BlockSpec((1, T, D), lambda bi, ti: (bi, ti, 0)),
        scratch_shapes=[pltpu.VMEM((1, M), jnp.float32)],
        compiler_params=pltpu.CompilerParams(
            dimension_semantics=("parallel", "arbitrary"),
        ),
        interpret=interpret,
    )(x, Wc, bc, WoA, Wq, bq)
    return out


def kernel(x, Wr, br, Ww, bw, Wt, bt, Wp, bp, Wo, bo):
    return _run(x, Wr, br, Ww, bw, Wt, bt, Wp, bp, Wo, bo)


# speedup vs baseline: 21.4694x; 1.3860x over previous
import functools

import jax
import jax.numpy as jnp
from jax.experimental import pallas as pl
from jax.experimental.pallas import tpu as pltpu

_T = 512


def _fold_body(wp_ref, wob_ref, bp_ref, bo_ref, wq_ref, bq_ref):
    wob = wob_ref[...]
    wq_ref[...] = jnp.dot(wp_ref[...], wob, preferred_element_type=jnp.float32)
    bq_ref[...] = (
        jnp.dot(bp_ref[...], wob, preferred_element_type=jnp.float32) + bo_ref[...]
    )


def _main_body(x_ref, wc_ref, bc_ref, woa_ref, wq_ref, bq_ref, o_ref, m_ref):
    M = wq_ref.shape[0]
    T = x_ref.shape[1]

    @pl.when(pl.program_id(1) == 0)
    def _():
        m_ref[...] = jnp.zeros_like(m_ref)

    xb = x_ref[0]
    g = jnp.dot(xb, wc_ref[...], preferred_element_type=jnp.float32) + bc_ref[...]
    gT = jnp.transpose(g)
    rw = jax.nn.sigmoid(gT[:M])
    ww = jax.nn.sigmoid(gT[M:2 * M])
    nm = gT[2 * M:3 * M]

    a = 1.0 - ww
    b = ww * nm

    A, Bv = a, b
    k = 1
    while k < T:
        pad1 = jnp.ones((M, k), jnp.float32)
        pad0 = jnp.zeros((M, k), jnp.float32)
        A_sh = jnp.concatenate([pad1, A[:, : T - k]], axis=1)
        B_sh = jnp.concatenate([pad0, Bv[:, : T - k]], axis=1)
        Bv = A * B_sh + Bv
        A = A * A_sh
        k *= 2

    m_in = m_ref[:, :1]
    A_prev = jnp.concatenate([jnp.ones((M, 1), jnp.float32), A[:, : T - 1]], axis=1)
    B_prev = jnp.concatenate([jnp.zeros((M, 1), jnp.float32), Bv[:, : T - 1]], axis=1)
    rm = jnp.transpose(rw * (A_prev * m_in + B_prev))
    m_ref[:, :1] = A[:, T - 1:] * m_in + Bv[:, T - 1:]

    h = jnp.dot(xb, woa_ref[...], preferred_element_type=jnp.float32)
    h = h + jnp.dot(rm, wq_ref[...], preferred_element_type=jnp.float32)
    o_ref[0] = jnp.tanh(h + bq_ref[...])


@functools.partial(jax.jit, static_argnames=("interpret",))
def _run(x, Wr, br, Ww, bw, Wt, bt, Wp, bp, Wo, bo, interpret=False):
    B, S, D = x.shape
    M = Wr.shape[1]
    T = _T

    Wc = jnp.concatenate(
        [Wr, Ww, Wt, jnp.zeros((D, M), jnp.float32)], axis=1)
    bc = jnp.concatenate([br, bw, bt, jnp.zeros((M,), jnp.float32)]).reshape(1, 4 * M)
    WoA = Wo[:D]
    WoB = Wo[D:]

    Wq, bq = pl.pallas_call(
        _fold_body,
        out_shape=(
            jax.ShapeDtypeStruct((M, D), jnp.float32),
            jax.ShapeDtypeStruct((1, D), jnp.float32),
        ),
        interpret=interpret,
    )(Wp, WoB, bp.reshape(1, D), bo.reshape(1, D))

    out = pl.pallas_call(
        _main_body,
        out_shape=jax.ShapeDtypeStruct((B, S, D), jnp.float32),
        grid=(B, S // T),
        in_specs=[
            pl.BlockSpec((1, T, D), lambda bi, ti: (bi, ti, 0)),
            pl.BlockSpec((D, 4 * M), lambda bi, ti: (0, 0)),
            pl.BlockSpec((1, 4 * M), lambda bi, ti: (0, 0)),
            pl.BlockSpec((D, D), lambda bi, ti: (0, 0)),
            pl.BlockSpec((M, D), lambda bi, ti: (0, 0)),
            pl.BlockSpec((1, D), lambda bi, ti: (0, 0)),
        ],
        out_specs=pl.BlockSpec((1, T, D), lambda bi, ti: (bi, ti, 0)),
        scratch_shapes=[pltpu.VMEM((M, 128), jnp.float32)],
        compiler_params=pltpu.CompilerParams(
            dimension_semantics=("parallel", "arbitrary"),
        ),
        interpret=interpret,
    )(x, Wc, bc, WoA, Wq, bq)
    return out


def kernel(x, Wr, br, Ww, bw, Wt, bt, Wp, bp, Wo, bo):
    return _run(x, Wr, br, Ww, bw, Wt, bt, Wp, bp, Wo, bo)


# speedup vs baseline: 22.1856x; 1.0334x over previous
import functools

import jax
import jax.numpy as jnp
from jax.experimental import pallas as pl
from jax.experimental.pallas import tpu as pltpu

_T = 512


def _fold_body(wp_ref, wob_ref, bp_ref, bo_ref, wq_ref, bq_ref):
    wob = wob_ref[...]
    wq_ref[...] = jnp.dot(wp_ref[...], wob, preferred_element_type=jnp.float32)
    bq_ref[...] = (
        jnp.dot(bp_ref[...], wob, preferred_element_type=jnp.float32) + bo_ref[...]
    )


def _main_body(x_ref, wc_ref, bc_ref, woa_ref, wq_ref, bq_ref, o_ref, m_ref):
    M = wq_ref.shape[0]
    T = x_ref.shape[1]

    @pl.when(pl.program_id(1) == 0)
    def _():
        m_ref[...] = jnp.zeros_like(m_ref)

    xb = x_ref[0].astype(jnp.bfloat16)
    g = jnp.dot(xb, wc_ref[...], preferred_element_type=jnp.float32) + bc_ref[...]
    gT = jnp.transpose(g)
    rw = jax.nn.sigmoid(gT[:M])
    ww = jax.nn.sigmoid(gT[M:2 * M])
    nm = gT[2 * M:3 * M]

    a = 1.0 - ww
    b = ww * nm

    def shift(X, k, fill):
        if k >= T:
            return jnp.full((M, T), fill, jnp.float32)
        pad = jnp.full((M, k), fill, jnp.float32)
        return jnp.concatenate([pad, X[:, : T - k]], axis=1)

    A, Bv = a, b
    k = 1
    while k < T:
        A1, B1 = shift(A, k, 1.0), shift(Bv, k, 0.0)
        A2, B2 = shift(A, 2 * k, 1.0), shift(Bv, 2 * k, 0.0)
        A3, B3 = shift(A, 3 * k, 1.0), shift(Bv, 3 * k, 0.0)
        Bv = Bv + A * (B1 + A1 * (B2 + A2 * B3))
        A = A * (A1 * (A2 * A3))
        k *= 4

    m_in = m_ref[:, :1]
    A_prev = jnp.concatenate([jnp.ones((M, 1), jnp.float32), A[:, : T - 1]], axis=1)
    B_prev = jnp.concatenate([jnp.zeros((M, 1), jnp.float32), Bv[:, : T - 1]], axis=1)
    rmT = rw * (A_prev * m_in + B_prev)
    m_ref[:, :1] = A[:, T - 1:] * m_in + Bv[:, T - 1:]

    h = (jnp.dot(xb, woa_ref[...], preferred_element_type=jnp.float32)
         + jnp.einsum("mt,md->td", rmT.astype(jnp.bfloat16), wq_ref[...],
                      preferred_element_type=jnp.float32))
    o_ref[0] = jnp.tanh(h + bq_ref[...])


@functools.partial(jax.jit, static_argnames=("interpret",))
def _run(x, Wr, br, Ww, bw, Wt, bt, Wp, bp, Wo, bo, interpret=False):
    B, S, D = x.shape
    M = Wr.shape[1]
    T = _T

    Wc = jnp.concatenate(
        [Wr, Ww, Wt, jnp.zeros((D, M), jnp.float32)], axis=1
    ).astype(jnp.bfloat16)
    bc = jnp.concatenate([br, bw, bt, jnp.zeros((M,), jnp.float32)]).reshape(1, 4 * M)
    WoA = Wo[:D].astype(jnp.bfloat16)
    WoB = Wo[D:]

    Wq, bq = pl.pallas_call(
        _fold_body,
        out_shape=(
            jax.ShapeDtypeStruct((M, D), jnp.float32),
            jax.ShapeDtypeStruct((1, D), jnp.float32),
        ),
        interpret=interpret,
    )(Wp, WoB, bp.reshape(1, D), bo.reshape(1, D))

    Wq = Wq.astype(jnp.bfloat16)

    out = pl.pallas_call(
        _main_body,
        out_shape=jax.ShapeDtypeStruct((B, S, D), jnp.float32),
        grid=(B, S // T),
        in_specs=[
            pl.BlockSpec((1, T, D), lambda bi, ti: (bi, ti, 0)),
            pl.BlockSpec((D, 4 * M), lambda bi, ti: (0, 0)),
            pl.BlockSpec((1, 4 * M), lambda bi, ti: (0, 0)),
            pl.BlockSpec((D, D), lambda bi, ti: (0, 0)),
            pl.BlockSpec((M, D), lambda bi, ti: (0, 0)),
            pl.BlockSpec((1, D), lambda bi, ti: (0, 0)),
        ],
        out_specs=pl.BlockSpec((1, T, D), lambda bi, ti: (bi, ti, 0)),
        scratch_shapes=[pltpu.VMEM((M, 128), jnp.float32)],
        compiler_params=pltpu.CompilerParams(
            dimension_semantics=("parallel", "arbitrary"),
            fuse_transposed_lhs_in_matmul=True,
        ),
        interpret=interpret,
    )(x, Wc, bc, WoA, Wq, bq)
    return out


def kernel(x, Wr, br, Ww, bw, Wt, bt, Wp, bp, Wo, bo):
    return _run(x, Wr, br, Ww, bw, Wt, bt, Wp, bp, Wo, bo)


# speedup vs baseline: 24.3205x; 1.0962x over previous
import functools

import jax
import jax.numpy as jnp
from jax.experimental import pallas as pl
from jax.experimental.pallas import tpu as pltpu

_T = 1024


def _fold_body(wp_ref, wob_ref, bp_ref, bo_ref, wq_ref, bq_ref):
    wob = wob_ref[...]
    wq_ref[...] = jnp.dot(wp_ref[...], wob, preferred_element_type=jnp.float32)
    bq_ref[...] = (
        jnp.dot(bp_ref[...], wob, preferred_element_type=jnp.float32) + bo_ref[...]
    )


def _main_body(x_ref, wc_ref, bc_ref, woa_ref, wq_ref, bq_ref, o_ref, m_ref):
    M = wq_ref.shape[0]
    T = x_ref.shape[1]

    @pl.when(pl.program_id(1) == 0)
    def _():
        m_ref[...] = jnp.zeros_like(m_ref)

    xb = x_ref[0].astype(jnp.bfloat16)
    g = jnp.dot(xb, wc_ref[...], preferred_element_type=jnp.float32) + bc_ref[...]
    gT = jnp.transpose(g)
    rw = jax.nn.sigmoid(gT[:M])
    ww = jax.nn.sigmoid(gT[M:2 * M])
    nm = gT[2 * M:3 * M]

    a = 1.0 - ww
    b = ww * nm

    def shift(X, k, fill):
        if k >= T:
            return jnp.full((M, T), fill, jnp.float32)
        pad = jnp.full((M, k), fill, jnp.float32)
        return jnp.concatenate([pad, X[:, : T - k]], axis=1)

    A, Bv = a, b
    k = 1
    while k < T:
        A1, B1 = shift(A, k, 1.0), shift(Bv, k, 0.0)
        A2, B2 = shift(A, 2 * k, 1.0), shift(Bv, 2 * k, 0.0)
        A3, B3 = shift(A, 3 * k, 1.0), shift(Bv, 3 * k, 0.0)
        Bv = Bv + A * (B1 + A1 * (B2 + A2 * B3))
        A = A * (A1 * (A2 * A3))
        k *= 4

    m_in = m_ref[:, :1]
    A_prev = jnp.concatenate([jnp.ones((M, 1), jnp.float32), A[:, : T - 1]], axis=1)
    B_prev = jnp.concatenate([jnp.zeros((M, 1), jnp.float32), Bv[:, : T - 1]], axis=1)
    rmT = rw * (A_prev * m_in + B_prev)
    m_ref[:, :1] = A[:, T - 1:] * m_in + Bv[:, T - 1:]

    h = (jnp.dot(xb, woa_ref[...], preferred_element_type=jnp.float32)
         + jnp.einsum("mt,md->td", rmT.astype(jnp.bfloat16), wq_ref[...],
                      preferred_element_type=jnp.float32))
    o_ref[0] = jnp.tanh(h + bq_ref[...])


@functools.partial(jax.jit, static_argnames=("interpret",))
def _run(x, Wr, br, Ww, bw, Wt, bt, Wp, bp, Wo, bo, interpret=False):
    B, S, D = x.shape
    M = Wr.shape[1]
    T = _T

    Wc = jnp.concatenate(
        [Wr, Ww, Wt, jnp.zeros((D, M), jnp.float32)], axis=1
    ).astype(jnp.bfloat16)
    bc = jnp.concatenate([br, bw, bt, jnp.zeros((M,), jnp.float32)]).reshape(1, 4 * M)
    WoA = Wo[:D].astype(jnp.bfloat16)
    WoB = Wo[D:]

    Wq, bq = pl.pallas_call(
        _fold_body,
        out_shape=(
            jax.ShapeDtypeStruct((M, D), jnp.float32),
            jax.ShapeDtypeStruct((1, D), jnp.float32),
        ),
        interpret=interpret,
    )(Wp, WoB, bp.reshape(1, D), bo.reshape(1, D))

    Wq = Wq.astype(jnp.bfloat16)

    out = pl.pallas_call(
        _main_body,
        out_shape=jax.ShapeDtypeStruct((B, S, D), jnp.float32),
        grid=(B, S // T),
        in_specs=[
            pl.BlockSpec((1, T, D), lambda bi, ti: (bi, ti, 0)),
            pl.BlockSpec((D, 4 * M), lambda bi, ti: (0, 0)),
            pl.BlockSpec((1, 4 * M), lambda bi, ti: (0, 0)),
            pl.BlockSpec((D, D), lambda bi, ti: (0, 0)),
            pl.BlockSpec((M, D), lambda bi, ti: (0, 0)),
            pl.BlockSpec((1, D), lambda bi, ti: (0, 0)),
        ],
        out_specs=pl.BlockSpec((1, T, D), lambda bi, ti: (bi, ti, 0)),
        scratch_shapes=[pltpu.VMEM((M, 128), jnp.float32)],
        compiler_params=pltpu.CompilerParams(
            dimension_semantics=("parallel", "arbitrary"),
            fuse_transposed_lhs_in_matmul=True,
        ),
        interpret=interpret,
    )(x, Wc, bc, WoA, Wq, bq)
    return out


def kernel(x, Wr, br, Ww, bw, Wt, bt, Wp, bp, Wo, bo):
    return _run(x, Wr, br, Ww, bw, Wt, bt, Wp, bp, Wo, bo)


# speedup vs baseline: 25.1599x; 1.0345x over previous
import functools

import jax
import jax.numpy as jnp
from jax.experimental import pallas as pl
from jax.experimental.pallas import tpu as pltpu

_T = 2048


def _fold_body(wp_ref, wob_ref, bp_ref, bo_ref, wq_ref, bq_ref):
    wob = wob_ref[...]
    wq_ref[...] = jnp.dot(wp_ref[...], wob, preferred_element_type=jnp.float32)
    bq_ref[...] = (
        jnp.dot(bp_ref[...], wob, preferred_element_type=jnp.float32) + bo_ref[...]
    )


def _main_body(x_ref, wc_ref, bc_ref, woa_ref, wq_ref, bq_ref, o_ref, m_ref):
    M = wq_ref.shape[0]
    T = x_ref.shape[1]

    @pl.when(pl.program_id(1) == 0)
    def _():
        m_ref[...] = jnp.zeros_like(m_ref)

    xb = x_ref[0].astype(jnp.bfloat16)
    g = jnp.dot(xb, wc_ref[...], preferred_element_type=jnp.float32) + bc_ref[...]
    gT = jnp.transpose(g)
    rw = jax.nn.sigmoid(gT[:M])
    ww = jax.nn.sigmoid(gT[M:2 * M])
    nm = gT[2 * M:3 * M]

    a = 1.0 - ww
    b = ww * nm

    def shift(X, k, fill):
        if k >= T:
            return jnp.full((M, T), fill, jnp.float32)
        pad = jnp.full((M, k), fill, jnp.float32)
        return jnp.concatenate([pad, X[:, : T - k]], axis=1)

    A, Bv = a, b
    k = 1
    while k < T:
        A1, B1 = shift(A, k, 1.0), shift(Bv, k, 0.0)
        A2, B2 = shift(A, 2 * k, 1.0), shift(Bv, 2 * k, 0.0)
        A3, B3 = shift(A, 3 * k, 1.0), shift(Bv, 3 * k, 0.0)
        Bv = Bv + A * (B1 + A1 * (B2 + A2 * B3))
        A = A * (A1 * (A2 * A3))
        k *= 4

    m_in = m_ref[:, :1]
    A_prev = jnp.concatenate([jnp.ones((M, 1), jnp.float32), A[:, : T - 1]], axis=1)
    B_prev = jnp.concatenate([jnp.zeros((M, 1), jnp.float32), Bv[:, : T - 1]], axis=1)
    rmT = rw * (A_prev * m_in + B_prev)
    m_ref[:, :1] = A[:, T - 1:] * m_in + Bv[:, T - 1:]

    h = (jnp.dot(xb, woa_ref[...], preferred_element_type=jnp.float32)
         + jnp.einsum("mt,md->td", rmT.astype(jnp.bfloat16), wq_ref[...],
                      preferred_element_type=jnp.float32))
    o_ref[0] = jnp.tanh(h + bq_ref[...])


@functools.partial(jax.jit, static_argnames=("interpret",))
def _run(x, Wr, br, Ww, bw, Wt, bt, Wp, bp, Wo, bo, interpret=False):
    B, S, D = x.shape
    M = Wr.shape[1]
    T = _T

    Wc = jnp.concatenate(
        [Wr, Ww, Wt, jnp.zeros((D, M), jnp.float32)], axis=1
    ).astype(jnp.bfloat16)
    bc = jnp.concatenate([br, bw, bt, jnp.zeros((M,), jnp.float32)]).reshape(1, 4 * M)
    WoA = Wo[:D].astype(jnp.bfloat16)
    WoB = Wo[D:]

    Wq, bq = pl.pallas_call(
        _fold_body,
        out_shape=(
            jax.ShapeDtypeStruct((M, D), jnp.float32),
            jax.ShapeDtypeStruct((1, D), jnp.float32),
        ),
        interpret=interpret,
    )(Wp, WoB, bp.reshape(1, D), bo.reshape(1, D))

    Wq = Wq.astype(jnp.bfloat16)

    out = pl.pallas_call(
        _main_body,
        out_shape=jax.ShapeDtypeStruct((B, S, D), jnp.float32),
        grid=(B, S // T),
        in_specs=[
            pl.BlockSpec((1, T, D), lambda bi, ti: (bi, ti, 0)),
            pl.BlockSpec((D, 4 * M), lambda bi, ti: (0, 0)),
            pl.BlockSpec((1, 4 * M), lambda bi, ti: (0, 0)),
            pl.BlockSpec((D, D), lambda bi, ti: (0, 0)),
            pl.BlockSpec((M, D), lambda bi, ti: (0, 0)),
            pl.BlockSpec((1, D), lambda bi, ti: (0, 0)),
        ],
        out_specs=pl.BlockSpec((1, T, D), lambda bi, ti: (bi, ti, 0)),
        scratch_shapes=[pltpu.VMEM((M, 128), jnp.float32)],
        compiler_params=pltpu.CompilerParams(
            dimension_semantics=("parallel", "arbitrary"),
            fuse_transposed_lhs_in_matmul=True,
        ),
        interpret=interpret,
    )(x, Wc, bc, WoA, Wq, bq)
    return out


def kernel(x, Wr, br, Ww, bw, Wt, bt, Wp, bp, Wo, bo):
    return _run(x, Wr, br, Ww, bw, Wt, bt, Wp, bp, Wo, bo)


# speedup vs baseline: 26.2141x; 1.0419x over previous
import functools

import jax
import jax.numpy as jnp
from jax.experimental import pallas as pl
from jax.experimental.pallas import tpu as pltpu

_T = 2048


def _fold_body(woa_ref, wob_ref, wp_ref, bp_ref, bo_ref,
               woa_bf_ref, wq_ref, bq_ref):
    woa_bf_ref[...] = woa_ref[...].astype(jnp.bfloat16)
    wob = wob_ref[...]
    wq_ref[...] = jnp.dot(
        wp_ref[...], wob, preferred_element_type=jnp.float32
    ).astype(jnp.bfloat16)
    bq_ref[...] = (
        jnp.dot(bp_ref[...], wob, preferred_element_type=jnp.float32) + bo_ref[...]
    )


def _main_body(x_ref, wc_ref, bc_ref, woa_ref, wq_ref, bq_ref, o_ref, m_ref):
    M = wq_ref.shape[0]
    T = x_ref.shape[1]

    @pl.when(pl.program_id(1) == 0)
    def _():
        m_ref[...] = jnp.zeros_like(m_ref)

    xb = x_ref[0].astype(jnp.bfloat16)
    g = jnp.dot(xb, wc_ref[...], preferred_element_type=jnp.float32) + bc_ref[...]
    gT = jnp.transpose(g)
    rw = jax.nn.sigmoid(gT[:M])
    ww = jax.nn.sigmoid(gT[M:2 * M])
    nm = gT[2 * M:3 * M]

    a = 1.0 - ww
    b = ww * nm

    def shift(X, k, fill):
        if k >= T:
            return jnp.full((M, T), fill, jnp.float32)
        pad = jnp.full((M, k), fill, jnp.float32)
        return jnp.concatenate([pad, X[:, : T - k]], axis=1)

    A, Bv = a, b
    k = 1
    while k < T:
        A1, B1 = shift(A, k, 1.0), shift(Bv, k, 0.0)
        A2, B2 = shift(A, 2 * k, 1.0), shift(Bv, 2 * k, 0.0)
        A3, B3 = shift(A, 3 * k, 1.0), shift(Bv, 3 * k, 0.0)
        Bv = Bv + A * (B1 + A1 * (B2 + A2 * B3))
        A = A * (A1 * (A2 * A3))
        k *= 4

    m_in = m_ref[:, :1]
    A_prev = jnp.concatenate([jnp.ones((M, 1), jnp.float32), A[:, : T - 1]], axis=1)
    B_prev = jnp.concatenate([jnp.zeros((M, 1), jnp.float32), Bv[:, : T - 1]], axis=1)
    rmT = rw * (A_prev * m_in + B_prev)
    m_ref[:, :1] = A[:, T - 1:] * m_in + Bv[:, T - 1:]

    h = (jnp.dot(xb, woa_ref[...], preferred_element_type=jnp.float32)
         + jnp.einsum("mt,md->td", rmT.astype(jnp.bfloat16), wq_ref[...],
                      preferred_element_type=jnp.float32))
    o_ref[0] = jnp.tanh(h + bq_ref[...])


@functools.partial(jax.jit, static_argnames=("interpret",))
def _run(x, Wr, br, Ww, bw, Wt, bt, Wp, bp, Wo, bo, interpret=False):
    B, S, D = x.shape
    M = Wr.shape[1]
    T = _T

    Wc = jnp.concatenate(
        [Wr, Ww, Wt, jnp.zeros((D, M), jnp.float32)], axis=1
    ).astype(jnp.bfloat16)
    bc = jnp.concatenate([br, bw, bt, jnp.zeros((M,), jnp.float32)]).reshape(1, 4 * M)

    WoA, Wq, bq = pl.pallas_call(
        _fold_body,
        out_shape=(
            jax.ShapeDtypeStruct((D, D), jnp.bfloat16),
            jax.ShapeDtypeStruct((M, D), jnp.bfloat16),
            jax.ShapeDtypeStruct((1, D), jnp.float32),
        ),
        grid=(1,),
        in_specs=[
            pl.BlockSpec((D, D), lambda i: (0, 0)),
            pl.BlockSpec((D, D), lambda i: (1, 0)),
            pl.BlockSpec((M, D), lambda i: (0, 0)),
            pl.BlockSpec((1, D), lambda i: (0, 0)),
            pl.BlockSpec((1, D), lambda i: (0, 0)),
        ],
        out_specs=(
            pl.BlockSpec((D, D), lambda i: (0, 0)),
            pl.BlockSpec((M, D), lambda i: (0, 0)),
            pl.BlockSpec((1, D), lambda i: (0, 0)),
        ),
        interpret=interpret,
    )(Wo, Wo, Wp, bp.reshape(1, D), bo.reshape(1, D))

    out = pl.pallas_call(
        _main_body,
        out_shape=jax.ShapeDtypeStruct((B, S, D), jnp.float32),
        grid=(B, S // T),
        in_specs=[
            pl.BlockSpec((1, T, D), lambda bi, ti: (bi, ti, 0)),
            pl.BlockSpec((D, 4 * M), lambda bi, ti: (0, 0)),
            pl.BlockSpec((1, 4 * M), lambda bi, ti: (0, 0)),
            pl.BlockSpec((D, D), lambda bi, ti: (0, 0)),
            pl.BlockSpec((M, D), lambda bi, ti: (0, 0)),
            pl.BlockSpec((1, D), lambda bi, ti: (0, 0)),
        ],
        out_specs=pl.BlockSpec((1, T, D), lambda bi, ti: (bi, ti, 0)),
        scratch_shapes=[pltpu.VMEM((M, 128), jnp.float32)],
        compiler_params=pltpu.CompilerParams(
            dimension_semantics=("parallel", "arbitrary"),
            fuse_transposed_lhs_in_matmul=True,
        ),
        interpret=interpret,
    )(x, Wc, bc, WoA, Wq, bq)
    return out


def kernel(x, Wr, br, Ww, bw, Wt, bt, Wp, bp, Wo, bo):
    return _run(x, Wr, br, Ww, bw, Wt, bt, Wp, bp, Wo, bo)


# speedup vs baseline: 27.8574x; 1.0627x over previous
import functools

import jax
import jax.numpy as jnp
from jax.experimental import pallas as pl
from jax.experimental.pallas import tpu as pltpu

_T = 2048


def _main_body(x_ref, wc_ref, bc_ref, woa_ref, wob_ref, wp_ref, bpo_ref,
               o_ref, m_ref):
    M = wp_ref.shape[0]
    T = x_ref.shape[1]

    @pl.when(pl.program_id(1) == 0)
    def _():
        m_ref[...] = jnp.zeros_like(m_ref)

    xb = x_ref[0].astype(jnp.bfloat16)
    g = jnp.dot(xb, wc_ref[...], preferred_element_type=jnp.float32) + bc_ref[...]
    gT = jnp.transpose(g)
    rw = jax.nn.sigmoid(gT[:M])
    ww = jax.nn.sigmoid(gT[M:2 * M])
    nm = gT[2 * M:3 * M]

    a = 1.0 - ww
    b = ww * nm

    def shift(X, k, fill):
        if k >= T:
            return jnp.full((M, T), fill, jnp.float32)
        pad = jnp.full((M, k), fill, jnp.float32)
        return jnp.concatenate([pad, X[:, : T - k]], axis=1)

    A, Bv = a, b
    k = 1
    while k < T:
        A1, B1 = shift(A, k, 1.0), shift(Bv, k, 0.0)
        A2, B2 = shift(A, 2 * k, 1.0), shift(Bv, 2 * k, 0.0)
        A3, B3 = shift(A, 3 * k, 1.0), shift(Bv, 3 * k, 0.0)
        Bv = Bv + A * (B1 + A1 * (B2 + A2 * B3))
        A = A * (A1 * (A2 * A3))
        k *= 4

    m_in = m_ref[:, :1]
    A_prev = jnp.concatenate([jnp.ones((M, 1), jnp.float32), A[:, : T - 1]], axis=1)
    B_prev = jnp.concatenate([jnp.zeros((M, 1), jnp.float32), Bv[:, : T - 1]], axis=1)
    rmT = rw * (A_prev * m_in + B_prev)
    m_ref[:, :1] = A[:, T - 1:] * m_in + Bv[:, T - 1:]

    wob = wob_ref[...].astype(jnp.bfloat16)
    wq = jnp.dot(wp_ref[...].astype(jnp.bfloat16), wob,
                 preferred_element_type=jnp.float32).astype(jnp.bfloat16)
    bq = (jnp.dot(bpo_ref[:1].astype(jnp.bfloat16), wob,
                  preferred_element_type=jnp.float32) + bpo_ref[1:])

    h = (jnp.dot(xb, woa_ref[...].astype(jnp.bfloat16),
                 preferred_element_type=jnp.float32)
         + jnp.einsum("mt,md->td", rmT.astype(jnp.bfloat16), wq,
                      preferred_element_type=jnp.float32))
    o_ref[0] = jnp.tanh(h + bq)


@functools.partial(jax.jit, static_argnames=("interpret",))
def _run(x, Wr, br, Ww, bw, Wt, bt, Wp, bp, Wo, bo, interpret=False):
    B, S, D = x.shape
    M = Wr.shape[1]
    T = _T

    Wc = jnp.concatenate(
        [Wr, Ww, Wt, jnp.zeros((D, M), jnp.float32)], axis=1
    ).astype(jnp.bfloat16)
    bc = jnp.concatenate([br, bw, bt, jnp.zeros((M,), jnp.float32)]).reshape(1, 4 * M)
    bpo = jnp.stack([bp, bo])

    out = pl.pallas_call(
        _main_body,
        out_shape=jax.ShapeDtypeStruct((B, S, D), jnp.float32),
        grid=(B, S // T),
        in_specs=[
            pl.BlockSpec((1, T, D), lambda bi, ti: (bi, ti, 0)),
            pl.BlockSpec((D, 4 * M), lambda bi, ti: (0, 0)),
            pl.BlockSpec((1, 4 * M), lambda bi, ti: (0, 0)),
            pl.BlockSpec((D, D), lambda bi, ti: (0, 0)),
            pl.BlockSpec((D, D), lambda bi, ti: (1, 0)),
            pl.BlockSpec((M, D), lambda bi, ti: (0, 0)),
            pl.BlockSpec((2, D), lambda bi, ti: (0, 0)),
        ],
        out_specs=pl.BlockSpec((1, T, D), lambda bi, ti: (bi, ti, 0)),
        scratch_shapes=[pltpu.VMEM((M, 128), jnp.float32)],
        compiler_params=pltpu.CompilerParams(
            dimension_semantics=("parallel", "arbitrary"),
            fuse_transposed_lhs_in_matmul=True,
        ),
        interpret=interpret,
    )(x, Wc, bc, Wo, Wo, Wp, bpo)
    return out


def kernel(x, Wr, br, Ww, bw, Wt, bt, Wp, bp, Wo, bo):
    return _run(x, Wr, br, Ww, bw, Wt, bt, Wp, bp, Wo, bo)


# speedup vs baseline: 27.9665x; 1.0039x over previous
import functools

import jax
import jax.numpy as jnp
from jax.experimental import pallas as pl
from jax.experimental.pallas import tpu as pltpu

_T = 2048


def _main_body(x_ref, wc_ref, bc_ref, woa_ref, wob_ref, wp_ref, bpo_ref,
               o_ref, m_ref, *, single_chunk):
    M = wp_ref.shape[0]
    T = x_ref.shape[1]

    if not single_chunk:
        @pl.when(pl.program_id(1) == 0)
        def _():
            m_ref[...] = jnp.zeros_like(m_ref)

    xb = x_ref[0].astype(jnp.bfloat16)
    g = jnp.dot(xb, wc_ref[...], preferred_element_type=jnp.float32) + bc_ref[...]
    gT = jnp.transpose(g)
    rw = jax.nn.sigmoid(gT[:M])
    ww = jax.nn.sigmoid(gT[M:2 * M])
    nm = gT[2 * M:3 * M]

    a = 1.0 - ww
    b = ww * nm

    def shift(X, k, fill):
        if k >= T:
            return jnp.full((M, T), fill, jnp.float32)
        pad = jnp.full((M, k), fill, jnp.float32)
        return jnp.concatenate([pad, X[:, : T - k]], axis=1)

    A, Bv = a, b
    k = 1
    while k < T:
        A1, B1 = shift(A, k, 1.0), shift(Bv, k, 0.0)
        A2, B2 = shift(A, 2 * k, 1.0), shift(Bv, 2 * k, 0.0)
        A3, B3 = shift(A, 3 * k, 1.0), shift(Bv, 3 * k, 0.0)
        Bv = Bv + A * (B1 + A1 * (B2 + A2 * B3))
        A = A * (A1 * (A2 * A3))
        k *= 4

    B_prev = jnp.concatenate([jnp.zeros((M, 1), jnp.float32), Bv[:, : T - 1]], axis=1)
    if single_chunk:
        rmT = rw * B_prev
    else:
        m_in = m_ref[:, :1]
        A_prev = jnp.concatenate(
            [jnp.ones((M, 1), jnp.float32), A[:, : T - 1]], axis=1)
        rmT = rw * (A_prev * m_in + B_prev)
        m_ref[:, :1] = A[:, T - 1:] * m_in + Bv[:, T - 1:]

    wob = wob_ref[...].astype(jnp.bfloat16)
    wq = jnp.dot(wp_ref[...].astype(jnp.bfloat16), wob,
                 preferred_element_type=jnp.float32).astype(jnp.bfloat16)
    bq = (jnp.dot(bpo_ref[:1].astype(jnp.bfloat16), wob,
                  preferred_element_type=jnp.float32) + bpo_ref[1:])

    h = (jnp.dot(xb, woa_ref[...].astype(jnp.bfloat16),
                 preferred_element_type=jnp.float32)
         + jnp.einsum("mt,md->td", rmT.astype(jnp.bfloat16), wq,
                      preferred_element_type=jnp.float32))
    o_ref[0] = jnp.tanh(h + bq)


@functools.partial(jax.jit, static_argnames=("interpret",))
def _run(x, Wr, br, Ww, bw, Wt, bt, Wp, bp, Wo, bo, interpret=False):
    B, S, D = x.shape
    M = Wr.shape[1]
    T = _T

    Wc = jnp.concatenate(
        [Wr, Ww, Wt, jnp.zeros((D, M), jnp.float32)], axis=1
    ).astype(jnp.bfloat16)
    bc = jnp.concatenate([br, bw, bt, jnp.zeros((M,), jnp.float32)]).reshape(1, 4 * M)
    bpo = jnp.stack([bp, bo])

    out = pl.pallas_call(
        functools.partial(_main_body, single_chunk=(T == S)),
        out_shape=jax.ShapeDtypeStruct((B, S, D), jnp.float32),
        grid=(B, S // T),
        in_specs=[
            pl.BlockSpec((1, T, D), lambda bi, ti: (bi, ti, 0)),
            pl.BlockSpec((D, 4 * M), lambda bi, ti: (0, 0)),
            pl.BlockSpec((1, 4 * M), lambda bi, ti: (0, 0)),
            pl.BlockSpec((D, D), lambda bi, ti: (0, 0)),
            pl.BlockSpec((D, D), lambda bi, ti: (1, 0)),
            pl.BlockSpec((M, D), lambda bi, ti: (0, 0)),
            pl.BlockSpec((2, D), lambda bi, ti: (0, 0)),
        ],
        out_specs=pl.BlockSpec((1, T, D), lambda bi, ti: (bi, ti, 0)),
        scratch_shapes=[pltpu.VMEM((M, 128), jnp.float32)],
        compiler_params=pltpu.CompilerParams(
            dimension_semantics=("parallel", "arbitrary"),
            fuse_transposed_lhs_in_matmul=True,
        ),
        interpret=interpret,
    )(x, Wc, bc, Wo, Wo, Wp, bpo)
    return out


def kernel(x, Wr, br, Ww, bw, Wt, bt, Wp, bp, Wo, bo):
    return _run(x, Wr, br, Ww, bw, Wt, bt, Wp, bp, Wo, bo)
